# baseline (speedup 1.0000x reference)
import functools

import jax
import jax.numpy as jnp
from jax import lax
from jax.experimental import pallas as pl
from jax.experimental.pallas import tpu as pltpu

EPS_ = 1e-10
CHUNK_R = 8
NBUF = 16


def _loss_kernel(row_ref, col_ref, bnd_ref, rm_hbm, out_ref,
                 bufs, sems, num_acc, den_acc, *, h, w, nb, s):
    cpb = s // CHUNK_R
    nchunk = nb * cpb

    def chunk_src(c):
        b = c // cpb
        s0 = (c % cpb) * CHUNK_R
        return rm_hbm.at[b, pl.ds(s0, CHUNK_R)]

    for k in range(NBUF):
        pltpu.make_async_copy(chunk_src(k), bufs.at[k], sems.at[k]).start()

    num_acc[0] = 0.0
    den_acc[0] = 0.0

    def body(c, _):
        slot = c % NBUF
        pltpu.make_async_copy(chunk_src(c), bufs.at[slot], sems.at[slot]).wait()

        x = bufs[slot]
        b = c // cpb
        row = row_ref[c, 0]
        col = col_ref[c, 0]
        bmap = bnd_ref[b, 0]

        iota_w = lax.broadcasted_iota(jnp.int32, (CHUNK_R, 1, w), 2)
        mask_w = iota_w == col[:, None, None]
        iota_h = lax.broadcasted_iota(jnp.int32, (CHUNK_R, h), 1)
        mask_h = iota_h == row[:, None]

        sum_w = jnp.sum(x, axis=2)
        denom = jnp.sum(sum_w, axis=1)

        srm_w = jnp.sum(jnp.where(mask_w, x, 0.0), axis=2)
        srm = jnp.sum(jnp.where(mask_h, srm_w, 0.0), axis=1)

        sb_w = jnp.sum(jnp.where(mask_w, bmap[None], 0.0), axis=2)
        sb = jnp.sum(jnp.where(mask_h, sb_w, 0.0), axis=1)

        num_acc[0] += jnp.sum(sb * -jnp.log(EPS_ + srm / denom))
        den_acc[0] += jnp.sum(sb)

        @pl.when(c + NBUF < nchunk)
        def _refill():
            pltpu.make_async_copy(chunk_src(c + NBUF), bufs.at[slot],
                                  sems.at[slot]).start()
        return _

    lax.fori_loop(0, nchunk, body, 0)
    out_ref[...] = jnp.full((1, 1), num_acc[0] / (1.0 + den_acc[0]), jnp.float32)


def kernel(response_map, source_feature_1d_locations, boundaries):
    B, S, H, W = response_map.shape
    CPB = S // CHUNK_R
    NCHUNK = B * CPB

    loc = source_feature_1d_locations.astype(jnp.int32)
    row = (loc // W).reshape(NCHUNK, 1, CHUNK_R)
    col = (loc % W).reshape(NCHUNK, 1, CHUNK_R)

    out = pl.pallas_call(
        functools.partial(_loss_kernel, h=H, w=W, nb=B, s=S),
        in_specs=[
            pl.BlockSpec(memory_space=pltpu.VMEM),
            pl.BlockSpec(memory_space=pltpu.VMEM),
            pl.BlockSpec(memory_space=pltpu.VMEM),
            pl.BlockSpec(memory_space=pl.ANY),
        ],
        out_specs=pl.BlockSpec(memory_space=pltpu.VMEM),
        out_shape=jax.ShapeDtypeStruct((1, 1), jnp.float32),
        scratch_shapes=[
            pltpu.VMEM((NBUF, CHUNK_R, H, W), jnp.float32),
            pltpu.SemaphoreType.DMA((NBUF,)),
            pltpu.SMEM((1,), jnp.float32),
            pltpu.SMEM((1,), jnp.float32),
        ],
    )(row, col, boundaries, response_map)
    return out[0, 0]


# speedup vs baseline: 1.1776x; 1.1776x over previous
import functools

import jax
import jax.numpy as jnp
from jax import lax
from jax.experimental import pallas as pl
from jax.experimental.pallas import tpu as pltpu

EPS_ = 1e-10
CHUNK_R = 32
NBUF = 8


def _loss_kernel(row_ref, col_ref, bnd_ref, rm_hbm, out_ref,
                 bufs, sems, num_acc, den_acc, *, h, w, nb, s):
    cpb = s // CHUNK_R
    nchunk = nb * cpb

    def chunk_src(c):
        b = c // cpb
        s0 = (c % cpb) * CHUNK_R
        return rm_hbm.at[b, pl.ds(s0, CHUNK_R)]

    for k in range(NBUF):
        pltpu.make_async_copy(chunk_src(k), bufs.at[k], sems.at[k]).start()

    num_acc[0] = 0.0
    den_acc[0] = 0.0

    def body(c, _):
        slot = c % NBUF
        pltpu.make_async_copy(chunk_src(c), bufs.at[slot], sems.at[slot]).wait()

        x = bufs[slot]
        b = c // cpb
        row = row_ref[c, 0]
        col = col_ref[c, 0]
        bmap = bnd_ref[b, 0]

        iota_w = lax.broadcasted_iota(jnp.int32, (CHUNK_R, 1, w), 2)
        mask_w = iota_w == col[:, None, None]
        iota_h = lax.broadcasted_iota(jnp.int32, (CHUNK_R, h), 1)
        mask_h = iota_h == row[:, None]

        sum_w = jnp.sum(x, axis=2)
        denom = jnp.sum(sum_w, axis=1)

        srm_w = jnp.sum(jnp.where(mask_w, x, 0.0), axis=2)
        srm = jnp.sum(jnp.where(mask_h, srm_w, 0.0), axis=1)

        sb_w = jnp.sum(jnp.where(mask_w, bmap[None], 0.0), axis=2)
        sb = jnp.sum(jnp.where(mask_h, sb_w, 0.0), axis=1)

        num_acc[0] += jnp.sum(sb * -jnp.log(EPS_ + srm / denom))
        den_acc[0] += jnp.sum(sb)

        @pl.when(c + NBUF < nchunk)
        def _refill():
            pltpu.make_async_copy(chunk_src(c + NBUF), bufs.at[slot],
                                  sems.at[slot]).start()
        return _

    lax.fori_loop(0, nchunk, body, 0)
    out_ref[...] = jnp.full((1, 1), num_acc[0] / (1.0 + den_acc[0]), jnp.float32)


def kernel(response_map, source_feature_1d_locations, boundaries):
    B, S, H, W = response_map.shape
    CPB = S // CHUNK_R
    NCHUNK = B * CPB

    loc = source_feature_1d_locations.astype(jnp.int32)
    row = (loc // W).reshape(NCHUNK, 1, CHUNK_R)
    col = (loc % W).reshape(NCHUNK, 1, CHUNK_R)

    out = pl.pallas_call(
        functools.partial(_loss_kernel, h=H, w=W, nb=B, s=S),
        in_specs=[
            pl.BlockSpec(memory_space=pltpu.VMEM),
            pl.BlockSpec(memory_space=pltpu.VMEM),
            pl.BlockSpec(memory_space=pltpu.VMEM),
            pl.BlockSpec(memory_space=pl.ANY),
        ],
        out_specs=pl.BlockSpec(memory_space=pltpu.VMEM),
        out_shape=jax.ShapeDtypeStruct((1, 1), jnp.float32),
        scratch_shapes=[
            pltpu.VMEM((NBUF, CHUNK_R, H, W), jnp.float32),
            pltpu.SemaphoreType.DMA((NBUF,)),
            pltpu.SMEM((1,), jnp.float32),
            pltpu.SMEM((1,), jnp.float32),
        ],
    )(row, col, boundaries, response_map)
    return out[0, 0]
